# baseline (speedup 1.0000x reference)
import functools

import jax
import jax.numpy as jnp
from jax import lax
from jax.experimental import pallas as pl
from jax.experimental.pallas import tpu as pltpu
from jax.experimental.pallas import tpu_sc as plsc

N_EXPERTS = 8
TOKENS = 4096
D_IN = 1024
D_OUT = 1024

BLK = 128
MAXB = TOKENS // BLK + N_EXPERTS - 1
PAD_ROWS = MAXB * BLK

NW = 32
ROWS_PER_W = TOKENS // NW
CHUNK = 64


def _sc_mesh():
    return plsc.VectorSubcoreMesh(core_axis_name="c", subcore_axis_name="s")


def _sc_scatter_body(src_hbm, dest_hbm, out_hbm, idx_v, rows_v, sem):
    wid = lax.axis_index("s") * 2 + lax.axis_index("c")
    base = wid * ROWS_PER_W
    for c in range(ROWS_PER_W // CHUNK):
        cb = base + c * CHUNK
        pltpu.sync_copy(dest_hbm.at[pl.ds(cb, CHUNK)], idx_v)
        pltpu.sync_copy(src_hbm.at[pl.ds(cb, CHUNK)], rows_v)
        pltpu.async_copy(rows_v, out_hbm.at[idx_v], sem).wait()


def _sc_scatter_rows(src, dest):
    kern = functools.partial(
        pl.kernel,
        out_type=jax.ShapeDtypeStruct((PAD_ROWS, D_IN), jnp.float32),
        mesh=_sc_mesh(),
        scratch_types=[
            pltpu.VMEM((CHUNK,), jnp.int32),
            pltpu.VMEM((CHUNK, D_IN), jnp.float32),
            pltpu.SemaphoreType.DMA,
        ],
    )(_sc_scatter_body)
    return kern(src, dest)


def _sc_gather_body(src_hbm, dest_hbm, out_hbm, idx_v, rows_v, sem):
    wid = lax.axis_index("s") * 2 + lax.axis_index("c")
    base = wid * ROWS_PER_W
    for c in range(ROWS_PER_W // CHUNK):
        cb = base + c * CHUNK
        pltpu.sync_copy(dest_hbm.at[pl.ds(cb, CHUNK)], idx_v)
        pltpu.async_copy(src_hbm.at[idx_v], rows_v, sem).wait()
        pltpu.sync_copy(rows_v, out_hbm.at[pl.ds(cb, CHUNK)])


def _sc_gather_rows(src, dest):
    kern = functools.partial(
        pl.kernel,
        out_type=jax.ShapeDtypeStruct((TOKENS, D_OUT), jnp.float32),
        mesh=_sc_mesh(),
        scratch_types=[
            pltpu.VMEM((CHUNK,), jnp.int32),
            pltpu.VMEM((CHUNK, D_OUT), jnp.float32),
            pltpu.SemaphoreType.DMA,
        ],
    )(_sc_gather_body)
    return kern(src, dest)


def _mm_body(meta_ref, xs_ref, w_ref, b_ref, out_ref):
    s = pl.program_id(0)
    nb = meta_ref[2, 0]

    @pl.when(s < nb)
    def _compute():
        y = lax.dot_general(
            xs_ref[...], w_ref[0], (((1,), (1,)), ((), ())),
            preferred_element_type=jnp.float32,
        )
        out_ref[...] = y + b_ref[0, 0, :][None, :]


def _grouped_matmul(x_pad, W, b, meta):
    grid_spec = pltpu.PrefetchScalarGridSpec(
        num_scalar_prefetch=1,
        grid=(MAXB,),
        in_specs=[
            pl.BlockSpec((BLK, D_IN), lambda s, m: (m[0, s], 0)),
            pl.BlockSpec((1, D_OUT, D_IN), lambda s, m: (m[1, s], 0, 0)),
            pl.BlockSpec((1, 1, D_OUT), lambda s, m: (m[1, s], 0, 0)),
        ],
        out_specs=pl.BlockSpec((BLK, D_OUT), lambda s, m: (m[0, s], 0)),
    )
    return pl.pallas_call(
        _mm_body,
        grid_spec=grid_spec,
        out_shape=jax.ShapeDtypeStruct((PAD_ROWS, D_OUT), jnp.float32),
    )(meta, x_pad, W, b.reshape(N_EXPERTS, 1, D_OUT))


def _routing_metadata(p):
    onehot = (p[:, None] == jnp.arange(N_EXPERTS, dtype=jnp.int32)[None, :])
    oh32 = onehot.astype(jnp.int32)
    counts = jnp.sum(oh32, axis=0)
    rank = jnp.sum((jnp.cumsum(oh32, axis=0) - 1) * oh32, axis=1)
    blocks_per_e = (counts + BLK - 1) // BLK
    pb = jnp.concatenate([jnp.zeros((1,), jnp.int32),
                          jnp.cumsum(blocks_per_e).astype(jnp.int32)])
    dest = pb[p] * BLK + rank
    nb = pb[N_EXPERTS]

    steps = jnp.arange(MAXB, dtype=jnp.int32)
    block_ids = jnp.minimum(steps, nb - 1)
    expert_ids = jnp.sum(
        (block_ids[:, None] >= pb[None, 1:]).astype(jnp.int32), axis=1)
    nbv = jnp.full((MAXB,), nb, dtype=jnp.int32)
    meta = jnp.stack([block_ids, expert_ids, nbv]).astype(jnp.int32)
    return dest.astype(jnp.int32), meta


def kernel(input, partitions, W, b):
    input_shape = input.shape
    x = input.reshape(-1, input_shape[-1])
    p = partitions.reshape(-1).astype(jnp.int32)

    dest, meta = _routing_metadata(p)
    x_pad = _sc_scatter_rows(x, dest)
    out_pad = _grouped_matmul(x_pad, W, b, meta)
    out = _sc_gather_rows(out_pad, dest)
    return out.reshape(tuple(input_shape[:-1]) + (W.shape[1],))


# speedup vs baseline: 1.1633x; 1.1633x over previous
import functools

import jax
import jax.numpy as jnp
from jax import lax
from jax.experimental import pallas as pl
from jax.experimental.pallas import tpu as pltpu
from jax.experimental.pallas import tpu_sc as plsc

N_EXPERTS = 8
TOKENS = 4096
D_IN = 1024
D_OUT = 1024

BLK = 256
MAXB = TOKENS // BLK + N_EXPERTS - 1
PAD_ROWS = MAXB * BLK

NW = 32
ROWS_PER_W = TOKENS // NW
CHUNK = 64


def _sc_mesh():
    return plsc.VectorSubcoreMesh(core_axis_name="c", subcore_axis_name="s")


def _sc_scatter_body(src_hbm, dest_hbm, out_hbm, idx_v, rows_v, sem):
    wid = lax.axis_index("s") * 2 + lax.axis_index("c")
    base = wid * ROWS_PER_W
    for c in range(ROWS_PER_W // CHUNK):
        cb = base + c * CHUNK
        pltpu.sync_copy(dest_hbm.at[pl.ds(cb, CHUNK)], idx_v)
        pltpu.sync_copy(src_hbm.at[pl.ds(cb, CHUNK)], rows_v)
        pltpu.async_copy(rows_v, out_hbm.at[idx_v], sem).wait()


def _sc_scatter_rows(src, dest):
    kern = functools.partial(
        pl.kernel,
        out_type=jax.ShapeDtypeStruct((PAD_ROWS, D_IN), jnp.float32),
        mesh=_sc_mesh(),
        scratch_types=[
            pltpu.VMEM((CHUNK,), jnp.int32),
            pltpu.VMEM((CHUNK, D_IN), jnp.float32),
            pltpu.SemaphoreType.DMA,
        ],
    )(_sc_scatter_body)
    return kern(src, dest)


def _sc_gather_body(src_hbm, dest_hbm, out_hbm, idx_v, rows_v, sem):
    wid = lax.axis_index("s") * 2 + lax.axis_index("c")
    base = wid * ROWS_PER_W
    for c in range(ROWS_PER_W // CHUNK):
        cb = base + c * CHUNK
        pltpu.sync_copy(dest_hbm.at[pl.ds(cb, CHUNK)], idx_v)
        pltpu.async_copy(src_hbm.at[idx_v], rows_v, sem).wait()
        pltpu.sync_copy(rows_v, out_hbm.at[pl.ds(cb, CHUNK)])


def _sc_gather_rows(src, dest):
    kern = functools.partial(
        pl.kernel,
        out_type=jax.ShapeDtypeStruct((TOKENS, D_OUT), jnp.float32),
        mesh=_sc_mesh(),
        scratch_types=[
            pltpu.VMEM((CHUNK,), jnp.int32),
            pltpu.VMEM((CHUNK, D_OUT), jnp.float32),
            pltpu.SemaphoreType.DMA,
        ],
    )(_sc_gather_body)
    return kern(src, dest)


def _mm_body(meta_ref, xs_ref, w_ref, b_ref, out_ref):
    s = pl.program_id(0)
    nb = meta_ref[2, 0]

    @pl.when(s < nb)
    def _compute():
        y = lax.dot_general(
            xs_ref[...], w_ref[0], (((1,), (1,)), ((), ())),
            preferred_element_type=jnp.float32,
        )
        out_ref[...] = y + b_ref[0, 0, :][None, :]


def _grouped_matmul(x_pad, W, b, meta):
    grid_spec = pltpu.PrefetchScalarGridSpec(
        num_scalar_prefetch=1,
        grid=(MAXB,),
        in_specs=[
            pl.BlockSpec((BLK, D_IN), lambda s, m: (m[0, s], 0)),
            pl.BlockSpec((1, D_OUT, D_IN), lambda s, m: (m[1, s], 0, 0)),
            pl.BlockSpec((1, 1, D_OUT), lambda s, m: (m[1, s], 0, 0)),
        ],
        out_specs=pl.BlockSpec((BLK, D_OUT), lambda s, m: (m[0, s], 0)),
    )
    return pl.pallas_call(
        _mm_body,
        grid_spec=grid_spec,
        out_shape=jax.ShapeDtypeStruct((PAD_ROWS, D_OUT), jnp.float32),
    )(meta, x_pad, W, b.reshape(N_EXPERTS, 1, D_OUT))


def _routing_metadata(p):
    onehot = (p[:, None] == jnp.arange(N_EXPERTS, dtype=jnp.int32)[None, :])
    oh32 = onehot.astype(jnp.int32)
    counts = jnp.sum(oh32, axis=0)
    rank = jnp.sum((jnp.cumsum(oh32, axis=0) - 1) * oh32, axis=1)
    blocks_per_e = (counts + BLK - 1) // BLK
    pb = jnp.concatenate([jnp.zeros((1,), jnp.int32),
                          jnp.cumsum(blocks_per_e).astype(jnp.int32)])
    dest = pb[p] * BLK + rank
    nb = pb[N_EXPERTS]

    steps = jnp.arange(MAXB, dtype=jnp.int32)
    block_ids = jnp.minimum(steps, nb - 1)
    expert_ids = jnp.sum(
        (block_ids[:, None] >= pb[None, 1:]).astype(jnp.int32), axis=1)
    nbv = jnp.full((MAXB,), nb, dtype=jnp.int32)
    meta = jnp.stack([block_ids, expert_ids, nbv]).astype(jnp.int32)
    return dest.astype(jnp.int32), meta


def kernel(input, partitions, W, b):
    input_shape = input.shape
    x = input.reshape(-1, input_shape[-1])
    p = partitions.reshape(-1).astype(jnp.int32)

    dest, meta = _routing_metadata(p)
    x_pad = _sc_scatter_rows(x, dest)
    out_pad = _grouped_matmul(x_pad, W, b, meta)
    out = _sc_gather_rows(out_pad, dest)
    return out.reshape(tuple(input_shape[:-1]) + (W.shape[1],))
